# baseline (speedup 1.0000x reference)
import functools

import jax
import jax.numpy as jnp
from jax import lax
from jax.experimental import pallas as pl
from jax.experimental.pallas import tpu as pltpu
from jax.experimental.pallas import tpu_sc as plsc

D = 256
NKX = D // 2 + 1
NIMG = 32
NPTS = NIMG * D * NKX
NVOX = D * D * NKX
HS = 512
HSZ = NVOX // HS
NW = 32
CHUNK = NPTS // NW
LANES = 16
NBINS = HS * LANES
SEG_PAD = HS * LANES
TRASH = NPTS + SEG_PAD
TRASH_SZ = 4096
NBLEN = TRASH + TRASH_SZ + 2048

HBLK = 2064
RBLK = 768
ABLK = 2048


_SC_PARAMS = pltpu.CompilerParams(needs_layout_passes=False)


def _mesh():
    return plsc.VectorSubcoreMesh(core_axis_name="c", subcore_axis_name="s")


def _wid():
    return lax.axis_index("s") * 2 + lax.axis_index("c")


def _prep_body(fr_ref, fi_ref, ctf_ref, neg_ref, syky_ref, sxkx_ref,
               vr_ref, vi_ref, cc_ref):
    fr = fr_ref[0]
    fi = fi_ref[0]
    ctf = ctf_ref[0]
    ph = (-2.0 * jnp.pi) * (syky_ref[0, 0][:, None] + sxkx_ref[0, 0][None, :])
    c = jnp.cos(ph)
    s = jnp.sin(ph)
    pr = (fr * c - fi * s) * ctf
    pi = (fr * s + fi * c) * ctf
    sign = 1.0 - 2.0 * neg_ref[0]
    vr_ref[0] = pr
    vi_ref[0] = pi * sign
    cc_ref[0] = ctf * ctf


def _prep(fr, fi, ctf, negf, syky, sxkx):
    blk = pl.BlockSpec((1, D, NKX), lambda b: (b, 0, 0))
    sblk = lambda n: pl.BlockSpec((1, 1, n), lambda b: (b, 0, 0))
    return pl.pallas_call(
        _prep_body,
        grid=(NIMG,),
        in_specs=[blk, blk, blk, blk, sblk(D), sblk(NKX)],
        out_specs=[blk, blk, blk],
        out_shape=[jax.ShapeDtypeStruct((NIMG, D, NKX), jnp.float32)] * 3,
    )(fr, fi, ctf, negf, syky.reshape(NIMG, 1, D), sxkx.reshape(NIMG, 1, NKX))


def _hist_body(idxf, table, ibuf, hist, sem):
    wid = _wid()
    lane = lax.iota(jnp.int32, LANES)
    zero16 = jnp.zeros((LANES,), jnp.int32)
    one16 = jnp.ones((LANES,), jnp.int32)

    def zero(i, carry):
        hist[pl.ds(i * 16, 16)] = zero16
        return carry

    lax.fori_loop(0, NBINS // 16, zero, 0)

    def blk(j, carry):
        pltpu.sync_copy(idxf.at[pl.ds(wid * CHUNK + j * HBLK, HBLK)], ibuf)
        for v in range(HBLK // 16):
            iv = ibuf[pl.ds(v * 16, 16)]
            m = iv < NVOX
            binc = lax.div(iv, HSZ) * LANES + lane
            plsc.addupdate_scatter(hist, [binc], one16, mask=m)
        return carry

    lax.fori_loop(0, CHUNK // HBLK, blk, 0)
    pltpu.sync_copy(hist, table.at[wid])


def _hist(idxf):
    k = functools.partial(
        pl.kernel,
        mesh=_mesh(),
        compiler_params=_SC_PARAMS,
        out_type=jax.ShapeDtypeStruct((NW, NBINS), jnp.int32),
        scratch_types=[
            pltpu.VMEM((HBLK,), jnp.int32),
            pltpu.VMEM((NBINS,), jnp.int32),
            pltpu.SemaphoreType.DMA,
        ],
    )
    return k(_hist_body)(idxf)


def _reorder_body(idxf, vr, vi, cc, base2, bidx, bvr, bvi, bcc,
                  nxt, ibuf, rbuf, vrbuf, vibuf, ccbuf, dbuf, sem):
    wid = _wid()
    lane = lax.iota(jnp.int32, LANES)
    pltpu.sync_copy(base2.at[wid], nxt)

    def blk(j, carry):
        off = wid * CHUNK + j * RBLK
        pltpu.sync_copy(idxf.at[pl.ds(off, RBLK)], ibuf)
        pltpu.sync_copy(vr.at[pl.ds(off, RBLK)], vrbuf)
        pltpu.sync_copy(vi.at[pl.ds(off, RBLK)], vibuf)
        pltpu.sync_copy(cc.at[pl.ds(off, RBLK)], ccbuf)
        for v in range(RBLK // 16):
            iv = ibuf[pl.ds(v * 16, 16)]
            m = iv < NVOX
            hs = lax.div(iv, HSZ)
            binc = hs * LANES + lane
            cur = plsc.load_gather(nxt, [binc], mask=m)
            plsc.store_scatter(nxt, [binc], cur + 1, mask=m)
            rbuf[pl.ds(v * 16, 16)] = iv - hs * HSZ
            tr = TRASH + ((j * RBLK + v * 16) & 4095) + lane
            dest = jnp.where(m, cur, tr)
            dbuf[v // 8, pl.ds((v % 8) * 16, 16)] = dest
        handles = []
        for q in range(RBLK // 128):
            idx_ref = dbuf.at[q]
            sl = pl.ds(q * 128, 128)
            handles.append(pltpu.async_copy(rbuf.at[sl], bidx.at[idx_ref], sem))
            handles.append(pltpu.async_copy(vrbuf.at[sl], bvr.at[idx_ref], sem))
            handles.append(pltpu.async_copy(vibuf.at[sl], bvi.at[idx_ref], sem))
            handles.append(pltpu.async_copy(ccbuf.at[sl], bcc.at[idx_ref], sem))
        for h in handles:
            h.wait()
        return carry

    lax.fori_loop(0, CHUNK // RBLK, blk, 0)


def _reorder(idxf, vr, vi, cc, base2):
    k = functools.partial(
        pl.kernel,
        mesh=_mesh(),
        compiler_params=_SC_PARAMS,
        out_type=[
            jax.ShapeDtypeStruct((NBLEN,), jnp.int32),
            jax.ShapeDtypeStruct((NBLEN,), jnp.float32),
            jax.ShapeDtypeStruct((NBLEN,), jnp.float32),
            jax.ShapeDtypeStruct((NBLEN,), jnp.float32),
        ],
        scratch_types=[
            pltpu.VMEM((NBINS,), jnp.int32),
            pltpu.VMEM((RBLK,), jnp.int32),
            pltpu.VMEM((RBLK,), jnp.int32),
            pltpu.VMEM((RBLK,), jnp.float32),
            pltpu.VMEM((RBLK,), jnp.float32),
            pltpu.VMEM((RBLK,), jnp.float32),
            pltpu.VMEM((RBLK // 128, 128), jnp.int32),
            pltpu.SemaphoreType.DMA,
        ],
    )
    return k(_reorder_body)(idxf, vr, vi, cc, base2)


def _accum_body(bidx, bvr, bvi, bcc, starts, lens, numflat, wts, csq,
                acr, aci, acw, acc, rbuf, vrbuf, vibuf, ccbuf, sbuf, lbuf, sem):
    wid = _wid()
    lane = lax.iota(jnp.int32, LANES)
    zerof = jnp.zeros((LANES,), jnp.float32)
    onef = jnp.ones((LANES,), jnp.float32)
    pltpu.sync_copy(starts, sbuf)
    pltpu.sync_copy(lens, lbuf)

    def rnd(r, carry):
        hs = r * NW + wid
        hsv = jnp.full((LANES,), hs, jnp.int32)
        start = jnp.max(plsc.load_gather(sbuf, [hsv]))
        seglen = jnp.max(plsc.load_gather(lbuf, [hsv]))

        def zero(i, c2):
            acr[pl.ds(i * 16, 16)] = zerof
            aci[pl.ds(i * 16, 16)] = zerof
            acw[pl.ds(i * 16, 16)] = zerof
            acc[pl.ds(i * 16, 16)] = zerof
            return c2

        lax.fori_loop(0, HSZ // 16, zero, 0)

        def blk(j, c2):
            gpos = pl.multiple_of(start + j * ABLK, 16)
            pltpu.sync_copy(bidx.at[pl.ds(gpos, ABLK)], rbuf)
            pltpu.sync_copy(bvr.at[pl.ds(gpos, ABLK)], vrbuf)
            pltpu.sync_copy(bvi.at[pl.ds(gpos, ABLK)], vibuf)
            pltpu.sync_copy(bcc.at[pl.ds(gpos, ABLK)], ccbuf)
            lim = jnp.full((LANES,), seglen - j * ABLK, jnp.int32)
            for v in range(ABLK // 16):
                m = (v * 16 + lane) < lim
                rel = rbuf[pl.ds(v * 16, 16)]
                plsc.addupdate_scatter(acr, [rel], vrbuf[pl.ds(v * 16, 16)], mask=m)
                plsc.addupdate_scatter(aci, [rel], vibuf[pl.ds(v * 16, 16)], mask=m)
                plsc.addupdate_scatter(acw, [rel], onef, mask=m)
                plsc.addupdate_scatter(acc, [rel], ccbuf[pl.ds(v * 16, 16)], mask=m)
            return c2

        nblk = lax.div(seglen + (ABLK - 1), ABLK)
        lax.fori_loop(0, nblk, blk, 0)
        hbase = hs * HSZ
        pltpu.sync_copy(acr, numflat.at[0, pl.ds(hbase, HSZ)])
        pltpu.sync_copy(aci, numflat.at[1, pl.ds(hbase, HSZ)])
        pltpu.sync_copy(acw, wts.at[pl.ds(hbase, HSZ)])
        pltpu.sync_copy(acc, csq.at[pl.ds(hbase, HSZ)])
        return carry

    lax.fori_loop(0, HS // NW, rnd, 0)


def _accum(bidx, bvr, bvi, bcc, starts, lens):
    k = functools.partial(
        pl.kernel,
        mesh=_mesh(),
        compiler_params=_SC_PARAMS,
        out_type=[
            jax.ShapeDtypeStruct((2, NVOX), jnp.float32),
            jax.ShapeDtypeStruct((NVOX,), jnp.float32),
            jax.ShapeDtypeStruct((NVOX,), jnp.float32),
        ],
        scratch_types=[
            pltpu.VMEM((HSZ,), jnp.float32),
            pltpu.VMEM((HSZ,), jnp.float32),
            pltpu.VMEM((HSZ,), jnp.float32),
            pltpu.VMEM((HSZ,), jnp.float32),
            pltpu.VMEM((ABLK,), jnp.int32),
            pltpu.VMEM((ABLK,), jnp.float32),
            pltpu.VMEM((ABLK,), jnp.float32),
            pltpu.VMEM((ABLK,), jnp.float32),
            pltpu.VMEM((520,), jnp.int32),
            pltpu.VMEM((520,), jnp.int32),
            pltpu.SemaphoreType.DMA,
        ],
    )
    return k(_accum_body)(bidx, bvr, bvi, bcc, starts, lens)


def kernel(imgs, ctf, rotMats, hwShiftAngs, numerator, weights, ctfsq):
    f = jnp.fft.rfftn(imgs, axes=(-2, -1))
    fr = jnp.real(f).astype(jnp.float32)
    fi = jnp.imag(f).astype(jnp.float32)
    ky = jnp.fft.fftfreq(D).astype(jnp.float32)
    kx = jnp.fft.rfftfreq(D).astype(jnp.float32)
    syky = hwShiftAngs[:, 0, None] * ky[None, :]
    sxkx = hwShiftAngs[:, 1, None] * kx[None, :]

    yc = (jnp.fft.fftfreq(D) * D).astype(jnp.float32)
    xc = jnp.arange(NKX, dtype=jnp.float32)
    gx = jnp.broadcast_to(xc[None, :], (D, NKX))
    gy = jnp.broadcast_to(yc[:, None], (D, NKX))
    gz = jnp.zeros((D, NKX), dtype=jnp.float32)
    grid = jnp.stack([gx, gy, gz], axis=-1)
    rot = jnp.einsum('bij,hwj->bhwi', rotMats, grid)
    neg = rot[..., 0] < 0
    rot = jnp.where(neg[..., None], -rot, rot)
    xi = jnp.round(rot[..., 0]).astype(jnp.int32)
    yi = jnp.round(rot[..., 1]).astype(jnp.int32)
    zi = jnp.round(rot[..., 2]).astype(jnp.int32)
    half = D // 2
    valid = (xi >= 0) & (xi < NKX) & (jnp.abs(yi) < half) & (jnp.abs(zi) < half)
    yi = jnp.mod(yi, D)
    zi = jnp.mod(zi, D)
    flat = (zi * D + yi) * NKX + xi
    idxf = jnp.where(valid, flat, NVOX).reshape(NPTS)

    vr, vi, cc = _prep(fr, fi, ctf, neg.astype(jnp.float32), syky, sxkx)
    vr = vr.reshape(NPTS)
    vi = vi.reshape(NPTS)
    cc = cc.reshape(NPTS)

    table = _hist(idxf)

    t = table.reshape(NW, HS, LANES).transpose(1, 0, 2)
    tot = t.sum(axis=(1, 2))
    ptot = ((tot + 15) // 16) * 16
    seg_start = jnp.concatenate([jnp.zeros((1,), jnp.int32),
                                 jnp.cumsum(ptot)[:-1].astype(jnp.int32)])
    tf = t.reshape(HS, NW * LANES)
    inner = jnp.cumsum(tf, axis=1).astype(jnp.int32) - tf
    base = seg_start[:, None] + inner
    base2 = (base.reshape(HS, NW, LANES).transpose(1, 0, 2)
             .reshape(NW, NBINS).astype(jnp.int32))
    starts = jnp.zeros((520,), jnp.int32).at[:HS].set(seg_start)
    lens = jnp.zeros((520,), jnp.int32).at[:HS].set(tot.astype(jnp.int32))

    bidx, bvr, bvi, bcc = _reorder(idxf, vr, vi, cc, base2)
    numflat, wts, csq = _accum(bidx, bvr, bvi, bcc, starts, lens)

    new_num = numflat.reshape(2, D, D, NKX)
    new_w = wts.reshape(D, D, NKX)
    new_c = csq.reshape(D, D, NKX)
    return new_num, new_w, new_c
